# baseline (speedup 1.0000x reference)
import jax
import jax.numpy as jnp
from jax import lax
from jax.experimental import pallas as pl
from jax.experimental.pallas import tpu as pltpu
from jax.experimental.pallas import tpu_sc as plsc

NENTITY = 100000
NRELATION = 1000
HIDDEN_DIM = 64
TIME_DIM = 32
REL_DIM = HIDDEN_DIM + TIME_DIM
GAMMA = 12.0
BATCH = 16384

NC = 2
NS = 16
L = 16
NW = NC * NS
BPW = BATCH // NW
CH = 128
NCHUNK = BPW // CH

_S = (9.999997070e-01, -1.666657720e-01, 8.332557998e-03,
      -1.981257224e-04, 2.704047332e-06, -2.053408008e-08)
_INV2PI = 0.15915494309189535
_TWOPI = 6.283185307179586
_RND = 12582912.0


def _sin16(x):
  t = x * jnp.float32(_INV2PI)
  k = (t + jnp.float32(_RND)) - jnp.float32(_RND)
  r = x - k * jnp.float32(_TWOPI)
  r2 = r * r
  p = jnp.float32(_S[5])
  for c in (_S[4], _S[3], _S[2], _S[1], _S[0]):
    p = p * r2 + jnp.float32(c)
  return p * r


def _score_kernel(heads, rels, tails, days, t1, rel, out,
                  hx0, hx1, tx0, tx1, rx0, rx1,
                  h0, h1, tt0, tt1, rr0, rr1, dayv, outv, sem0, sem1):
  wid = lax.axis_index("s") * NC + lax.axis_index("c")
  base_w = wid * BPW
  pltpu.sync_copy(days.at[pl.ds(base_w, BPW)], dayv)

  hx = (hx0, hx1)
  tx = (tx0, tx1)
  rx = (rx0, rx1)
  hrow = (h0, h1)
  trow = (tt0, tt1)
  rrow = (rr0, rr1)
  sems = (sem0, sem1)
  msk_hi = jnp.uint32(0xFFFF0000)

  def load_idx(c):
    b = base_w + c * CH
    pltpu.sync_copy(heads.at[pl.ds(b, CH)], hx[c % 2])
    pltpu.sync_copy(tails.at[pl.ds(b, CH)], tx[c % 2])
    pltpu.sync_copy(rels.at[pl.ds(b, CH)], rx[c % 2])

  def fire(c):
    p = c % 2
    return [
        pltpu.async_copy(t1.at[hx[p]], hrow[p], sems[p]),
        pltpu.async_copy(t1.at[tx[p]], trow[p], sems[p]),
        pltpu.async_copy(rel.at[rx[p]], rrow[p], sems[p]),
    ]

  def compute(c):
    p = c % 2
    hb, tb, rb = hrow[p], trow[p], rrow[p]
    lanes = lax.iota(jnp.int32, L)
    gdn = lax.GatherDimensionNumbers(
        offset_dims=(), collapsed_slice_dims=(0,), start_index_map=(0,))
    shuf = [(lanes ^ sh)[:, None] for sh in (8, 4, 2, 1)]
    bcast = [jnp.full((L, 1), j, jnp.int32) for j in range(L)]

    def sample_acc(i, dayg, j):
      day = lax.gather(dayg, bcast[j], gdn, (1,),
                       mode=lax.GatherScatterMode.PROMISE_IN_BOUNDS)
      acc = jnp.abs(hb[i, pl.ds(0, L)] + rb[i, pl.ds(0, L)]
                    - tb[i, pl.ds(0, L)])
      for k in range(1, HIDDEN_DIM // L):
        acc = acc + jnp.abs(hb[i, pl.ds(k * L, L)]
                            + rb[i, pl.ds(k * L, L)]
                            - tb[i, pl.ds(k * L, L)])
      hphi = lax.bitcast_convert_type(hb[i, pl.ds(96, L)], jnp.uint32)
      hamp = lax.bitcast_convert_type(hb[i, pl.ds(112, L)], jnp.uint32)
      tphi = lax.bitcast_convert_type(tb[i, pl.ds(96, L)], jnp.uint32)
      tamp = lax.bitcast_convert_type(tb[i, pl.ds(112, L)], jnp.uint32)
      for k in range(TIME_DIM // L):
        if k == 0:
          hp = lax.bitcast_convert_type(hphi << 16, jnp.float32)
          ha = lax.bitcast_convert_type(hamp << 16, jnp.float32)
          tp = lax.bitcast_convert_type(tphi << 16, jnp.float32)
          ta = lax.bitcast_convert_type(tamp << 16, jnp.float32)
        else:
          hp = lax.bitcast_convert_type(hphi & msk_hi, jnp.float32)
          ha = lax.bitcast_convert_type(hamp & msk_hi, jnp.float32)
          tp = lax.bitcast_convert_type(tphi & msk_hi, jnp.float32)
          ta = lax.bitcast_convert_type(tamp & msk_hi, jnp.float32)
        fsl = pl.ds(HIDDEN_DIM + k * L, L)
        hs = _sin16(day * hb[i, fsl] + hp) * ha
        ts = _sin16(day * tb[i, fsl] + tp) * ta
        acc = acc + jnp.abs(hs + rb[i, fsl] - ts)
      return acc

    rev4 = [0, 8, 4, 12, 2, 10, 6, 14, 1, 9, 5, 13, 3, 11, 7, 15]

    def group_body(g, _):
      dayg = dayv[pl.ds(c * CH + g * L, L)]
      accs = [sample_acc(g * L + rev4[j], dayg, rev4[j]) for j in range(L)]
      for si, sh in enumerate((8, 4, 2, 1)):
        nxt = []
        for a, b in zip(accs[::2], accs[1::2]):
          ab = a + lax.gather(a, shuf[si], gdn, (1,),
                              mode=lax.GatherScatterMode.PROMISE_IN_BOUNDS)
          bb = b + lax.gather(b, shuf[si], gdn, (1,),
                              mode=lax.GatherScatterMode.PROMISE_IN_BOUNDS)
          nxt.append(jnp.where((lanes & sh) == 0, ab, bb))
        accs = nxt
      outv[pl.ds(c * CH + g * L, L)] = jnp.float32(GAMMA) - accs[0]
      return 0

    lax.fori_loop(0, CH // L, group_body, 0)

  load_idx(0)
  pending = {0: fire(0)}
  for c in range(NCHUNK):
    if c + 1 < NCHUNK:
      load_idx(c + 1)
      pending[c + 1] = fire(c + 1)
    for cp in pending.pop(c):
      cp.wait()
    compute(c)

  pltpu.sync_copy(outv, out.at[pl.ds(base_w, BPW)])


_RC = 14336


def _rne_bf16(x):
  u = lax.bitcast_convert_type(x, jnp.uint32)
  return (u + jnp.uint32(0x7FFF) + ((u >> 16) & jnp.uint32(1))) >> 16


def _repack_kernel(ent_t, frq_t, phi_t, amp_t, t1_out):
  phi = phi_t[...]
  amp = amp_t[...]
  phi_pack = (_rne_bf16(phi[TIME_DIM // 2:, :]) << 16) | _rne_bf16(
      phi[:TIME_DIM // 2, :])
  amp_pack = (_rne_bf16(amp[TIME_DIM // 2:, :]) << 16) | _rne_bf16(
      amp[:TIME_DIM // 2, :])
  cat = jnp.concatenate([
      ent_t[...], frq_t[...],
      lax.bitcast_convert_type(phi_pack, jnp.float32),
      lax.bitcast_convert_type(amp_pack, jnp.float32),
  ], axis=0)
  t1_out[...] = cat.T


def _repack(ent_t, frq_t, phi_t, amp_t):
  grid = pl.cdiv(NENTITY, _RC)
  return pl.pallas_call(
      _repack_kernel,
      grid=(grid,),
      in_specs=[
          pl.BlockSpec((HIDDEN_DIM, _RC), lambda j: (0, j)),
          pl.BlockSpec((TIME_DIM, _RC), lambda j: (0, j)),
          pl.BlockSpec((TIME_DIM, _RC), lambda j: (0, j)),
          pl.BlockSpec((TIME_DIM, _RC), lambda j: (0, j)),
      ],
      out_specs=pl.BlockSpec((_RC, 128), lambda j: (j, 0)),
      out_shape=jax.ShapeDtypeStruct((NENTITY, 128), jnp.float32),
  )(ent_t, frq_t, phi_t, amp_t)


@jax.jit
def kernel(sample, entity_embedding, relation_embedding, d_frq_embedding,
           d_phi_embedding, d_amp_embedding):
  heads = sample[:, 0]
  rels = sample[:, 1]
  tails = sample[:, 2]
  days = sample[:, 3].astype(jnp.float32)

  t1 = _repack(entity_embedding.T, d_frq_embedding.T,
               d_phi_embedding.T, d_amp_embedding.T)

  mesh = plsc.VectorSubcoreMesh(core_axis_name="c", subcore_axis_name="s")
  score = pl.kernel(
      _score_kernel,
      out_type=jax.ShapeDtypeStruct((BATCH,), jnp.float32),
      mesh=mesh,
      compiler_params=pltpu.CompilerParams(use_tc_tiling_on_sc=False),
      scratch_types=[
          pltpu.VMEM((CH,), jnp.int32),
          pltpu.VMEM((CH,), jnp.int32),
          pltpu.VMEM((CH,), jnp.int32),
          pltpu.VMEM((CH,), jnp.int32),
          pltpu.VMEM((CH,), jnp.int32),
          pltpu.VMEM((CH,), jnp.int32),
          pltpu.VMEM((CH, 128), jnp.float32),
          pltpu.VMEM((CH, 128), jnp.float32),
          pltpu.VMEM((CH, 128), jnp.float32),
          pltpu.VMEM((CH, 128), jnp.float32),
          pltpu.VMEM((CH, REL_DIM), jnp.float32),
          pltpu.VMEM((CH, REL_DIM), jnp.float32),
          pltpu.VMEM((BPW,), jnp.float32),
          pltpu.VMEM((BPW,), jnp.float32),
          pltpu.SemaphoreType.DMA,
          pltpu.SemaphoreType.DMA,
      ],
  )(heads, rels, tails, days, t1, relation_embedding)
  return score.reshape(BATCH, 1)


# speedup vs baseline: 1.0083x; 1.0083x over previous
import jax
import jax.numpy as jnp
from jax import lax
from jax.experimental import pallas as pl
from jax.experimental.pallas import tpu as pltpu
from jax.experimental.pallas import tpu_sc as plsc

NENTITY = 100000
NRELATION = 1000
HIDDEN_DIM = 64
TIME_DIM = 32
REL_DIM = HIDDEN_DIM + TIME_DIM
GAMMA = 12.0
BATCH = 16384

NC = 2
NS = 16
L = 16
NW = NC * NS
BPW = BATCH // NW
CH = 128
NCHUNK = BPW // CH

_S = (9.999997070e-01, -1.666657720e-01, 8.332557998e-03,
      -1.981257224e-04, 2.704047332e-06, -2.053408008e-08)
_INV2PI = 0.15915494309189535
_TWOPI = 6.283185307179586
_RND = 12582912.0


def _sin16(x):
  t = x * jnp.float32(_INV2PI)
  k = (t + jnp.float32(_RND)) - jnp.float32(_RND)
  r = x - k * jnp.float32(_TWOPI)
  r2 = r * r
  p = jnp.float32(_S[5])
  for c in (_S[4], _S[3], _S[2], _S[1], _S[0]):
    p = p * r2 + jnp.float32(c)
  return p * r


def _score_kernel(heads, rels, tails, days, t1, rel, out,
                  hx0, hx1, tx0, tx1, rx0, rx1,
                  h0, h1, tt0, tt1, rr0, rr1, dayv, outv, sem0, sem1):
  wid = lax.axis_index("s") * NC + lax.axis_index("c")
  base_w = wid * BPW
  pltpu.sync_copy(days.at[pl.ds(base_w, BPW)], dayv)

  hx = (hx0, hx1)
  tx = (tx0, tx1)
  rx = (rx0, rx1)
  hrow = (h0, h1)
  trow = (tt0, tt1)
  rrow = (rr0, rr1)
  sems = (sem0, sem1)
  msk_hi = jnp.uint32(0xFFFF0000)

  def load_idx(c):
    b = base_w + c * CH
    pltpu.sync_copy(heads.at[pl.ds(b, CH)], hx[c % 2])
    pltpu.sync_copy(tails.at[pl.ds(b, CH)], tx[c % 2])
    pltpu.sync_copy(rels.at[pl.ds(b, CH)], rx[c % 2])

  def fire(c):
    p = c % 2
    return [
        pltpu.async_copy(t1.at[hx[p]], hrow[p], sems[p]),
        pltpu.async_copy(t1.at[tx[p]], trow[p], sems[p]),
        pltpu.async_copy(rel.at[rx[p]], rrow[p], sems[p]),
    ]

  def compute(c):
    p = c % 2
    hb, tb, rb = hrow[p], trow[p], rrow[p]
    lanes = lax.iota(jnp.int32, L)
    gdn = lax.GatherDimensionNumbers(
        offset_dims=(), collapsed_slice_dims=(0,), start_index_map=(0,))
    shuf = [(lanes ^ sh)[:, None] for sh in (8, 4, 2, 1)]
    bcast = [jnp.full((L, 1), j, jnp.int32) for j in range(L)]

    def sample_acc(i, dayg, j):
      day = lax.gather(dayg, bcast[j], gdn, (1,),
                       mode=lax.GatherScatterMode.PROMISE_IN_BOUNDS)
      acc = jnp.abs(hb[i, pl.ds(0, L)] + rb[i, pl.ds(0, L)]
                    - tb[i, pl.ds(0, L)])
      for k in range(1, HIDDEN_DIM // L):
        acc = acc + jnp.abs(hb[i, pl.ds(k * L, L)]
                            + rb[i, pl.ds(k * L, L)]
                            - tb[i, pl.ds(k * L, L)])
      hphi = lax.bitcast_convert_type(hb[i, pl.ds(96, L)], jnp.uint32)
      hamp = lax.bitcast_convert_type(hb[i, pl.ds(112, L)], jnp.uint32)
      tphi = lax.bitcast_convert_type(tb[i, pl.ds(96, L)], jnp.uint32)
      tamp = lax.bitcast_convert_type(tb[i, pl.ds(112, L)], jnp.uint32)
      for k in range(TIME_DIM // L):
        if k == 0:
          hp = lax.bitcast_convert_type(hphi << 16, jnp.float32)
          ha = lax.bitcast_convert_type(hamp << 16, jnp.float32)
          tp = lax.bitcast_convert_type(tphi << 16, jnp.float32)
          ta = lax.bitcast_convert_type(tamp << 16, jnp.float32)
        else:
          hp = lax.bitcast_convert_type(hphi & msk_hi, jnp.float32)
          ha = lax.bitcast_convert_type(hamp & msk_hi, jnp.float32)
          tp = lax.bitcast_convert_type(tphi & msk_hi, jnp.float32)
          ta = lax.bitcast_convert_type(tamp & msk_hi, jnp.float32)
        fsl = pl.ds(HIDDEN_DIM + k * L, L)
        hs = _sin16(day * hb[i, fsl] + hp) * ha
        ts = _sin16(day * tb[i, fsl] + tp) * ta
        acc = acc + jnp.abs(hs + rb[i, fsl] - ts)
      return acc

    rev4 = [0, 8, 4, 12, 2, 10, 6, 14, 1, 9, 5, 13, 3, 11, 7, 15]

    def group_body(g, _):
      dayg = dayv[pl.ds(c * CH + g * L, L)]
      accs = [sample_acc(g * L + rev4[j], dayg, rev4[j]) for j in range(L)]
      for si, sh in enumerate((8, 4, 2, 1)):
        nxt = []
        for a, b in zip(accs[::2], accs[1::2]):
          ab = a + lax.gather(a, shuf[si], gdn, (1,),
                              mode=lax.GatherScatterMode.PROMISE_IN_BOUNDS)
          bb = b + lax.gather(b, shuf[si], gdn, (1,),
                              mode=lax.GatherScatterMode.PROMISE_IN_BOUNDS)
          nxt.append(jnp.where((lanes & sh) == 0, ab, bb))
        accs = nxt
      outv[pl.ds(c * CH + g * L, L)] = jnp.float32(GAMMA) - accs[0]
      return 0

    lax.fori_loop(0, CH // L, group_body, 0)

  load_idx(0)
  pending = {0: fire(0)}
  for c in range(NCHUNK):
    if c + 1 < NCHUNK:
      load_idx(c + 1)
      pending[c + 1] = fire(c + 1)
    for cp in pending.pop(c):
      cp.wait()
    compute(c)

  pltpu.sync_copy(outv, out.at[pl.ds(base_w, BPW)])


_RC = 16384


def _rne_bf16(x):
  u = lax.bitcast_convert_type(x, jnp.uint32)
  return (u + jnp.uint32(0x7FFF) + ((u >> 16) & jnp.uint32(1))) >> 16


def _repack_kernel(ent_t, frq_t, phi_t, amp_t, t1_out):
  phi = phi_t[...]
  amp = amp_t[...]
  phi_pack = (_rne_bf16(phi[TIME_DIM // 2:, :]) << 16) | _rne_bf16(
      phi[:TIME_DIM // 2, :])
  amp_pack = (_rne_bf16(amp[TIME_DIM // 2:, :]) << 16) | _rne_bf16(
      amp[:TIME_DIM // 2, :])
  cat = jnp.concatenate([
      ent_t[...], frq_t[...],
      lax.bitcast_convert_type(phi_pack, jnp.float32),
      lax.bitcast_convert_type(amp_pack, jnp.float32),
  ], axis=0)
  t1_out[...] = cat.T


def _repack(ent_t, frq_t, phi_t, amp_t):
  grid = pl.cdiv(NENTITY, _RC)
  return pl.pallas_call(
      _repack_kernel,
      grid=(grid,),
      in_specs=[
          pl.BlockSpec((HIDDEN_DIM, _RC), lambda j: (0, j)),
          pl.BlockSpec((TIME_DIM, _RC), lambda j: (0, j)),
          pl.BlockSpec((TIME_DIM, _RC), lambda j: (0, j)),
          pl.BlockSpec((TIME_DIM, _RC), lambda j: (0, j)),
      ],
      out_specs=pl.BlockSpec((_RC, 128), lambda j: (j, 0)),
      out_shape=jax.ShapeDtypeStruct((NENTITY, 128), jnp.float32),
  )(ent_t, frq_t, phi_t, amp_t)


@jax.jit
def kernel(sample, entity_embedding, relation_embedding, d_frq_embedding,
           d_phi_embedding, d_amp_embedding):
  heads = sample[:, 0]
  rels = sample[:, 1]
  tails = sample[:, 2]
  days = sample[:, 3].astype(jnp.float32)

  t1 = _repack(entity_embedding.T, d_frq_embedding.T,
               d_phi_embedding.T, d_amp_embedding.T)

  mesh = plsc.VectorSubcoreMesh(core_axis_name="c", subcore_axis_name="s")
  score = pl.kernel(
      _score_kernel,
      out_type=jax.ShapeDtypeStruct((BATCH,), jnp.float32),
      mesh=mesh,
      compiler_params=pltpu.CompilerParams(use_tc_tiling_on_sc=False),
      scratch_types=[
          pltpu.VMEM((CH,), jnp.int32),
          pltpu.VMEM((CH,), jnp.int32),
          pltpu.VMEM((CH,), jnp.int32),
          pltpu.VMEM((CH,), jnp.int32),
          pltpu.VMEM((CH,), jnp.int32),
          pltpu.VMEM((CH,), jnp.int32),
          pltpu.VMEM((CH, 128), jnp.float32),
          pltpu.VMEM((CH, 128), jnp.float32),
          pltpu.VMEM((CH, 128), jnp.float32),
          pltpu.VMEM((CH, 128), jnp.float32),
          pltpu.VMEM((CH, REL_DIM), jnp.float32),
          pltpu.VMEM((CH, REL_DIM), jnp.float32),
          pltpu.VMEM((BPW,), jnp.float32),
          pltpu.VMEM((BPW,), jnp.float32),
          pltpu.SemaphoreType.DMA,
          pltpu.SemaphoreType.DMA,
      ],
  )(heads, rels, tails, days, t1, relation_embedding)
  return score.reshape(BATCH, 1)


# speedup vs baseline: 1.0330x; 1.0244x over previous
import jax
import jax.numpy as jnp
from jax import lax
from jax.experimental import pallas as pl
from jax.experimental.pallas import tpu as pltpu
from jax.experimental.pallas import tpu_sc as plsc

NENTITY = 100000
NRELATION = 1000
HIDDEN_DIM = 64
TIME_DIM = 32
REL_DIM = HIDDEN_DIM + TIME_DIM
GAMMA = 12.0
BATCH = 16384

NC = 2
NS = 16
L = 16
NW = NC * NS
BPW = BATCH // NW
CH = 128
NCHUNK = BPW // CH

_S = (6.283088486e+00, -4.133324916e+01, 8.140011884e+01,
      -7.467607215e+01, 3.316849207e+01)
_INV2PI = 0.15915494309189535
_RND = 12582912.0


def _sin_turns16(t):
  k = (t + jnp.float32(_RND)) - jnp.float32(_RND)
  u = t - k
  u2 = u * u
  p = jnp.float32(_S[4])
  for c in (_S[3], _S[2], _S[1], _S[0]):
    p = p * u2 + jnp.float32(c)
  return p * u


def _score_kernel(heads, rels, tails, days, t1, rel, out,
                  hx0, hx1, tx0, tx1, rx0, rx1,
                  h0, h1, tt0, tt1, rr0, rr1, dayv, outv, sem0, sem1):
  wid = lax.axis_index("s") * NC + lax.axis_index("c")
  base_w = wid * BPW
  pltpu.sync_copy(days.at[pl.ds(base_w, BPW)], dayv)

  hx = (hx0, hx1)
  tx = (tx0, tx1)
  rx = (rx0, rx1)
  hrow = (h0, h1)
  trow = (tt0, tt1)
  rrow = (rr0, rr1)
  sems = (sem0, sem1)
  msk_hi = jnp.uint32(0xFFFF0000)

  def load_idx(c):
    b = base_w + c * CH
    pltpu.sync_copy(heads.at[pl.ds(b, CH)], hx[c % 2])
    pltpu.sync_copy(tails.at[pl.ds(b, CH)], tx[c % 2])
    pltpu.sync_copy(rels.at[pl.ds(b, CH)], rx[c % 2])

  def fire(c):
    p = c % 2
    return [
        pltpu.async_copy(t1.at[hx[p]], hrow[p], sems[p]),
        pltpu.async_copy(t1.at[tx[p]], trow[p], sems[p]),
        pltpu.async_copy(rel.at[rx[p]], rrow[p], sems[p]),
    ]

  def compute(c):
    p = c % 2
    hb, tb, rb = hrow[p], trow[p], rrow[p]
    lanes = lax.iota(jnp.int32, L)
    gdn = lax.GatherDimensionNumbers(
        offset_dims=(), collapsed_slice_dims=(0,), start_index_map=(0,))
    shuf = [(lanes ^ sh)[:, None] for sh in (8, 4, 2, 1)]
    bcast = [jnp.full((L, 1), j, jnp.int32) for j in range(L)]

    def sample_acc(i, dayg, j):
      day = lax.gather(dayg, bcast[j], gdn, (1,),
                       mode=lax.GatherScatterMode.PROMISE_IN_BOUNDS)
      acc = jnp.abs(hb[i, pl.ds(0, L)] + rb[i, pl.ds(0, L)]
                    - tb[i, pl.ds(0, L)])
      for k in range(1, HIDDEN_DIM // L):
        acc = acc + jnp.abs(hb[i, pl.ds(k * L, L)]
                            + rb[i, pl.ds(k * L, L)]
                            - tb[i, pl.ds(k * L, L)])
      hphi = lax.bitcast_convert_type(hb[i, pl.ds(96, L)], jnp.uint32)
      hamp = lax.bitcast_convert_type(hb[i, pl.ds(112, L)], jnp.uint32)
      tphi = lax.bitcast_convert_type(tb[i, pl.ds(96, L)], jnp.uint32)
      tamp = lax.bitcast_convert_type(tb[i, pl.ds(112, L)], jnp.uint32)
      for k in range(TIME_DIM // L):
        if k == 0:
          hp = lax.bitcast_convert_type(hphi << 16, jnp.float32)
          ha = lax.bitcast_convert_type(hamp << 16, jnp.float32)
          tp = lax.bitcast_convert_type(tphi << 16, jnp.float32)
          ta = lax.bitcast_convert_type(tamp << 16, jnp.float32)
        else:
          hp = lax.bitcast_convert_type(hphi & msk_hi, jnp.float32)
          ha = lax.bitcast_convert_type(hamp & msk_hi, jnp.float32)
          tp = lax.bitcast_convert_type(tphi & msk_hi, jnp.float32)
          ta = lax.bitcast_convert_type(tamp & msk_hi, jnp.float32)
        fsl = pl.ds(HIDDEN_DIM + k * L, L)
        hs = _sin_turns16(day * hb[i, fsl] + hp) * ha
        ts = _sin_turns16(day * tb[i, fsl] + tp) * ta
        acc = acc + jnp.abs(hs + rb[i, fsl] - ts)
      return acc

    rev4 = [0, 8, 4, 12, 2, 10, 6, 14, 1, 9, 5, 13, 3, 11, 7, 15]

    def group_body(g, _):
      dayg = dayv[pl.ds(c * CH + g * L, L)]
      accs = [sample_acc(g * L + rev4[j], dayg, rev4[j]) for j in range(L)]
      for si, sh in enumerate((8, 4, 2, 1)):
        nxt = []
        for a, b in zip(accs[::2], accs[1::2]):
          ab = a + lax.gather(a, shuf[si], gdn, (1,),
                              mode=lax.GatherScatterMode.PROMISE_IN_BOUNDS)
          bb = b + lax.gather(b, shuf[si], gdn, (1,),
                              mode=lax.GatherScatterMode.PROMISE_IN_BOUNDS)
          nxt.append(jnp.where((lanes & sh) == 0, ab, bb))
        accs = nxt
      outv[pl.ds(c * CH + g * L, L)] = jnp.float32(GAMMA) - accs[0]
      return 0

    lax.fori_loop(0, CH // L, group_body, 0)

  load_idx(0)
  pending = {0: fire(0)}
  for c in range(NCHUNK):
    if c + 1 < NCHUNK:
      load_idx(c + 1)
      pending[c + 1] = fire(c + 1)
    for cp in pending.pop(c):
      cp.wait()
    compute(c)

  pltpu.sync_copy(outv, out.at[pl.ds(base_w, BPW)])


_RC = 16384


def _rne_bf16(x):
  u = lax.bitcast_convert_type(x, jnp.uint32)
  return (u + jnp.uint32(0x7FFF) + ((u >> 16) & jnp.uint32(1))) >> 16


def _repack_kernel(ent_t, frq_t, phi_t, amp_t, t1_out):
  phi = phi_t[...] * jnp.float32(_INV2PI)
  amp = amp_t[...]
  phi_pack = (_rne_bf16(phi[TIME_DIM // 2:, :]) << 16) | _rne_bf16(
      phi[:TIME_DIM // 2, :])
  amp_pack = (_rne_bf16(amp[TIME_DIM // 2:, :]) << 16) | _rne_bf16(
      amp[:TIME_DIM // 2, :])
  cat = jnp.concatenate([
      ent_t[...], frq_t[...] * jnp.float32(_INV2PI),
      lax.bitcast_convert_type(phi_pack, jnp.float32),
      lax.bitcast_convert_type(amp_pack, jnp.float32),
  ], axis=0)
  t1_out[...] = cat.T


def _repack(ent_t, frq_t, phi_t, amp_t):
  grid = pl.cdiv(NENTITY, _RC)
  return pl.pallas_call(
      _repack_kernel,
      grid=(grid,),
      in_specs=[
          pl.BlockSpec((HIDDEN_DIM, _RC), lambda j: (0, j)),
          pl.BlockSpec((TIME_DIM, _RC), lambda j: (0, j)),
          pl.BlockSpec((TIME_DIM, _RC), lambda j: (0, j)),
          pl.BlockSpec((TIME_DIM, _RC), lambda j: (0, j)),
      ],
      out_specs=pl.BlockSpec((_RC, 128), lambda j: (j, 0)),
      out_shape=jax.ShapeDtypeStruct((NENTITY, 128), jnp.float32),
  )(ent_t, frq_t, phi_t, amp_t)


@jax.jit
def kernel(sample, entity_embedding, relation_embedding, d_frq_embedding,
           d_phi_embedding, d_amp_embedding):
  heads = sample[:, 0]
  rels = sample[:, 1]
  tails = sample[:, 2]
  days = sample[:, 3].astype(jnp.float32)

  t1 = _repack(entity_embedding.T, d_frq_embedding.T,
               d_phi_embedding.T, d_amp_embedding.T)

  mesh = plsc.VectorSubcoreMesh(core_axis_name="c", subcore_axis_name="s")
  score = pl.kernel(
      _score_kernel,
      out_type=jax.ShapeDtypeStruct((BATCH,), jnp.float32),
      mesh=mesh,
      compiler_params=pltpu.CompilerParams(use_tc_tiling_on_sc=False),
      scratch_types=[
          pltpu.VMEM((CH,), jnp.int32),
          pltpu.VMEM((CH,), jnp.int32),
          pltpu.VMEM((CH,), jnp.int32),
          pltpu.VMEM((CH,), jnp.int32),
          pltpu.VMEM((CH,), jnp.int32),
          pltpu.VMEM((CH,), jnp.int32),
          pltpu.VMEM((CH, 128), jnp.float32),
          pltpu.VMEM((CH, 128), jnp.float32),
          pltpu.VMEM((CH, 128), jnp.float32),
          pltpu.VMEM((CH, 128), jnp.float32),
          pltpu.VMEM((CH, REL_DIM), jnp.float32),
          pltpu.VMEM((CH, REL_DIM), jnp.float32),
          pltpu.VMEM((BPW,), jnp.float32),
          pltpu.VMEM((BPW,), jnp.float32),
          pltpu.SemaphoreType.DMA,
          pltpu.SemaphoreType.DMA,
      ],
  )(heads, rels, tails, days, t1, relation_embedding)
  return score.reshape(BATCH, 1)
